# baseline (speedup 1.0000x reference)
import functools

import jax
import jax.numpy as jnp
from jax import lax
from jax.experimental import pallas as pl
from jax.experimental.pallas import tpu as pltpu
from jax.experimental.pallas import tpu_sc as plsc

_NUM_CORES = 2
_NUM_SUBCORES = 16
_NUM_WORKERS = _NUM_CORES * _NUM_SUBCORES


def _gather_kernel(n_per_worker, chunk, seq, table_hbm, idx_hbm, out_hbm,
                   idx_v, rows_v, sem_g0, sem_g1, sem_s0, sem_s1):
    wid = lax.axis_index("s") * _NUM_CORES + lax.axis_index("c")
    base = wid * n_per_worker
    num_chunks = n_per_worker // chunk
    batches_per_chunk = chunk // seq
    gather_sems = (sem_g0, sem_g1)
    store_sems = (sem_s0, sem_s1)

    pltpu.sync_copy(idx_hbm.at[pl.ds(base, n_per_worker)], idx_v)

    def store_chunk(buf, row_off, sem):
        b0 = row_off // seq
        copies = []
        for k in range(batches_per_chunk):
            copies.append(pltpu.async_copy(
                rows_v.at[buf, pl.ds(k * seq, seq)],
                out_hbm.at[b0 + k], sem))
        return copies

    store_copies = [None, None]
    prev = None
    for i in range(num_chunks):
        b = i & 1
        if store_copies[b] is not None:
            for cp in store_copies[b]:
                cp.wait()
        g = pltpu.async_copy(
            table_hbm.at[idx_v.at[pl.ds(i * chunk, chunk)]],
            rows_v.at[b], gather_sems[b])
        if prev is not None:
            pg, pb, poff = prev
            pg.wait()
            store_copies[pb] = store_chunk(pb, poff, store_sems[pb])
        prev = (g, b, base + i * chunk)
    pg, pb, poff = prev
    pg.wait()
    store_copies[pb] = store_chunk(pb, poff, store_sems[pb])
    for b in range(2):
        if store_copies[b] is not None:
            for cp in store_copies[b]:
                cp.wait()


@jax.jit
def kernel(x, table):
    batch, seq = x.shape
    vocab, dim = table.shape
    n = batch * seq
    assert n % _NUM_WORKERS == 0
    n_per_worker = n // _NUM_WORKERS
    chunk = 800
    assert n_per_worker % chunk == 0 and chunk % seq == 0

    idx = x.reshape(n).astype(jnp.int32)

    mesh = plsc.VectorSubcoreMesh(
        core_axis_name="c", subcore_axis_name="s",
        num_cores=_NUM_CORES, num_subcores=_NUM_SUBCORES)

    out = pl.kernel(
        functools.partial(_gather_kernel, n_per_worker, chunk, seq),
        out_type=jax.ShapeDtypeStruct((batch, seq, dim), jnp.float32),
        mesh=mesh,
        scratch_types=[
            pltpu.VMEM((n_per_worker,), jnp.int32),
            pltpu.VMEM((2, chunk, dim), jnp.float32),
            pltpu.SemaphoreType.DMA,
            pltpu.SemaphoreType.DMA,
            pltpu.SemaphoreType.DMA,
            pltpu.SemaphoreType.DMA,
        ],
        compiler_params=pltpu.CompilerParams(use_tc_tiling_on_sc=False),
    )(table, idx)

    return out
